# baseline (speedup 1.0000x reference)
import functools

import jax
import jax.numpy as jnp
from jax.experimental import pallas as pl


ALPHA = 33
C = ALPHA + 1


def _planes_body(tok_ref, eng_ref, out_ref):
    tok = tok_ref[...]
    for c in range(ALPHA):
        out_ref[c] = (tok == c).astype(jnp.float32)
    out_ref[ALPHA] = (eng_ref[...] <= -1.0).astype(jnp.float32)


@functools.partial(jax.jit, static_argnames=("block_rows",))
def _run(tokens, energy_scores, block_rows=32):
    nb, nt = tokens.shape
    outp = pl.pallas_call(
        _planes_body,
        grid=(nb // block_rows,),
        in_specs=[
            pl.BlockSpec((block_rows, nt), lambda i: (i, 0)),
            pl.BlockSpec((block_rows, nt), lambda i: (i, 0)),
        ],
        out_specs=pl.BlockSpec((C, block_rows, nt), lambda i: (0, i, 0)),
        out_shape=jax.ShapeDtypeStruct((C, nb, nt), jnp.float32),
    )(tokens, energy_scores)
    return jnp.transpose(outp, (1, 2, 0))


def kernel(tokens, energy_scores):
    return _run(tokens, energy_scores)
